# baseline (speedup 1.0000x reference)
import functools

import jax
import jax.numpy as jnp
from jax.experimental import pallas as pl
from jax.experimental.pallas import tpu as pltpu

N = 50000
NG = 512
H = 4
C = 64
K = 5
Q = 512
NB = 98
NP = NB * Q
CC = 256
NCH = NP // CC

_BIG = 1e10
_INF = float("inf")
_IBIG = 2**30


def _knn_kernel(blo_ref, bhi_ref, posq_ref, bcol_ref, pch_ref, br_ref,
                src_ref):
    b = pl.program_id(0)
    qs = b * Q
    lo = blo_ref[b]
    hi = bhi_ref[b]
    c0 = lo // CC
    c1 = (hi - 1) // CC + 1

    qp = posq_ref[...]
    qx = qp[:, 0:1]
    qy = qp[:, 1:2]
    qz = qp[:, 2:3]
    qx2 = qx * qx + qy * qy + qz * qz
    qb = bcol_ref[...]
    rowg = jax.lax.broadcasted_iota(jnp.int32, (Q, CC), 0) + qs
    coll = jax.lax.broadcasted_iota(jnp.int32, (Q, CC), 1)
    pos10 = jax.lax.broadcasted_iota(jnp.int32, (Q, 2 * K), 1)

    def body(c, carry):
        best_d, best_i = carry
        base = c * CC
        cp = pch_ref[pl.ds(c, 1)].reshape(8, CC)
        cb = br_ref[pl.ds(c, 1), :]
        cx = cp[0:1, :]
        cy = cp[1:2, :]
        cz = cp[2:3, :]
        cx2 = cx * cx + cy * cy + cz * cz
        dot = jnp.dot(qp, cp, preferred_element_type=jnp.float32)
        d = cx2 - 2.0 * dot + qx2
        gidx = coll + base
        d = jnp.where((qb != cb) | (rowg == gidx), _BIG, d)
        cds, cis = [], []
        for _ in range(K):
            m = jnp.min(d, axis=1, keepdims=True)
            am = jnp.min(jnp.where(d == m, gidx, _IBIG), axis=1, keepdims=True)
            cds.append(m)
            cis.append(am)
            d = jnp.where(gidx == am, _INF, d)
        cat_d = jnp.concatenate([best_d] + cds, axis=1)
        cat_i = jnp.concatenate([best_i] + cis, axis=1)
        nds, nis = [], []
        for _ in range(K):
            m = jnp.min(cat_d, axis=1, keepdims=True)
            ap = jnp.min(jnp.where(cat_d == m, pos10, _IBIG), axis=1,
                         keepdims=True)
            sel = pos10 == ap
            nds.append(m)
            nis.append(jnp.sum(jnp.where(sel, cat_i, 0), axis=1,
                               keepdims=True))
            cat_d = jnp.where(sel, _INF, cat_d)
        return (jnp.concatenate(nds, axis=1), jnp.concatenate(nis, axis=1))

    init = (jnp.full((Q, K), _INF, jnp.float32),
            jnp.zeros((Q, K), jnp.int32))
    _, best_i = jax.lax.fori_loop(c0, c1, body, init)
    src_ref[...] = best_i


def _embed_kernel(z_ref, emb_ref, out_ref):
    zc = z_ref[...]
    cols = jax.lax.broadcasted_iota(jnp.int32, (Q, 128), 1)
    oh = (zc == cols).astype(jnp.float32)
    out_ref[...] = jnp.dot(oh, emb_ref[...],
                           preferred_element_type=jnp.float32)


def _gat_kernel(blo_ref, bhi_ref, src_ref, xq_ref, x_ref, Wl_ref, bl_ref,
                Wr_ref, br_ref, att_ref, bias_ref, out_ref):
    b = pl.program_id(0)
    lo = blo_ref[b]
    hi = bhi_ref[b]
    c0 = lo // CC
    c1 = (hi - 1) // CC + 1

    coll = jax.lax.broadcasted_iota(jnp.int32, (Q, CC), 1)

    def body(c, xjr):
        base = c * CC
        xc = x_ref[pl.ds(base, CC), :]
        new = []
        for k in range(K):
            idx = src_ref[:, k:k + 1] - base
            oh = (idx == coll).astype(jnp.float32)
            new.append(xjr[k] + jnp.dot(oh, xc,
                                        preferred_element_type=jnp.float32))
        return tuple(new)

    init = tuple(jnp.zeros((Q, C), jnp.float32) for _ in range(K))
    xjr = jax.lax.fori_loop(c0, c1, body, init)

    xi = jnp.dot(xq_ref[...], Wr_ref[...],
                 preferred_element_type=jnp.float32) + br_ref[...]
    att = att_ref[...]
    xjs, alphas = [], []
    for k in range(K):
        xjk = jnp.dot(xjr[k], Wl_ref[...],
                      preferred_element_type=jnp.float32) + bl_ref[...]
        e = xi + xjk
        e = jnp.where(e >= 0, e, 0.2 * e)
        p = e * att
        alphas.append([jnp.sum(p[:, h * C:(h + 1) * C], axis=1,
                               keepdims=True) for h in range(H)])
        xjs.append(xjk)

    acc = jnp.zeros((Q, C), jnp.float32)
    for h in range(H):
        ah = [alphas[k][h] for k in range(K)]
        amax = ah[0]
        for k in range(1, K):
            amax = jnp.maximum(amax, ah[k])
        ex = [jnp.exp(ah[k] - amax) for k in range(K)]
        den = ex[0]
        for k in range(1, K):
            den = den + ex[k]
        den = den + 1e-16
        for k in range(K):
            acc = acc + (ex[k] / den) * xjs[k][:, h * C:(h + 1) * C]

    xnew = acc * jnp.float32(1.0 / H) + bias_ref[...]
    out_ref[...] = jnp.maximum(xnew, 0.0)


def _pool_kernel(x_ref, br_ref, fcW_ref, fcb_ref, gsum_ref, cnt_ref,
                 res_ref):
    b = pl.program_id(0)

    @pl.when(b == 0)
    def _():
        gsum_ref[...] = jnp.zeros_like(gsum_ref)
        cnt_ref[...] = jnp.zeros_like(cnt_ref)

    brow = br_ref[0]
    gi = jax.lax.broadcasted_iota(jnp.int32, (NG, Q), 0)
    oh = (gi == brow).astype(jnp.float32)
    gsum_ref[...] += jnp.dot(oh, x_ref[...],
                             preferred_element_type=jnp.float32)
    cnt_ref[...] += jnp.sum(oh, axis=1, keepdims=True)

    @pl.when(b == NB - 1)
    def _():
        g = gsum_ref[...] / jnp.maximum(cnt_ref[...], 1.0)
        res_ref[...] = jnp.dot(g, fcW_ref[...],
                               preferred_element_type=jnp.float32) \
            + fcb_ref[...]


def kernel(z, pos, batch, emb, Wl0, bl0, Wr0, br0, att0, bias0,
           Wl1, bl1, Wr1, br1, att1, bias1,
           Wl2, bl2, Wr2, br2, att2, bias2, fc_W, fc_b):
    batch = batch.astype(jnp.int32)
    z = z.astype(jnp.int32)

    pad = NP - N
    posq = jnp.pad(pos, ((0, pad), (0, 5)))
    batp = jnp.pad(batch, (0, pad), constant_values=NG)
    bcol = batp[:, None]
    br3 = batp.reshape(NB, 1, Q)
    brch = batp.reshape(NCH, CC)
    pch = jnp.pad(pos, ((0, pad), (0, 5))).T.reshape(8, NCH, CC) \
        .transpose(1, 0, 2)
    zcol = jnp.pad(z, (0, pad))[:, None]

    ss = jnp.searchsorted(batch, jnp.arange(NG + 1)).astype(jnp.int32)
    firsts = jnp.minimum(jnp.arange(NB) * Q, N - 1)
    lasts = jnp.minimum(jnp.arange(NB) * Q + Q - 1, N - 1)
    blo = ss[batch[firsts]]
    bhi = ss[batch[lasts] + 1]

    knn_spec = pltpu.PrefetchScalarGridSpec(
        num_scalar_prefetch=2,
        grid=(NB,),
        in_specs=[
            pl.BlockSpec((Q, 8), lambda b, lo, hi: (b, 0)),
            pl.BlockSpec((Q, 1), lambda b, lo, hi: (b, 0)),
            pl.BlockSpec((NCH, 8, CC), lambda b, lo, hi: (0, 0, 0)),
            pl.BlockSpec((NCH, CC), lambda b, lo, hi: (0, 0)),
        ],
        out_specs=pl.BlockSpec((Q, K), lambda b, lo, hi: (b, 0)),
    )
    src = pl.pallas_call(
        _knn_kernel,
        grid_spec=knn_spec,
        out_shape=jax.ShapeDtypeStruct((NP, K), jnp.int32),
    )(blo, bhi, posq, bcol, pch, brch)

    embp = jnp.zeros((128, C), jnp.float32).at[:100].set(emb)
    x = pl.pallas_call(
        _embed_kernel,
        grid=(NB,),
        in_specs=[
            pl.BlockSpec((Q, 1), lambda b: (b, 0)),
            pl.BlockSpec((128, C), lambda b: (0, 0)),
        ],
        out_specs=pl.BlockSpec((Q, C), lambda b: (b, 0)),
        out_shape=jax.ShapeDtypeStruct((NP, C), jnp.float32),
    )(zcol, embp)

    gat_spec = pltpu.PrefetchScalarGridSpec(
        num_scalar_prefetch=2,
        grid=(NB,),
        in_specs=[
            pl.BlockSpec((Q, K), lambda b, lo, hi: (b, 0)),
            pl.BlockSpec((Q, C), lambda b, lo, hi: (b, 0)),
            pl.BlockSpec((NP, C), lambda b, lo, hi: (0, 0)),
            pl.BlockSpec((C, H * C), lambda b, lo, hi: (0, 0)),
            pl.BlockSpec((1, H * C), lambda b, lo, hi: (0, 0)),
            pl.BlockSpec((C, H * C), lambda b, lo, hi: (0, 0)),
            pl.BlockSpec((1, H * C), lambda b, lo, hi: (0, 0)),
            pl.BlockSpec((1, H * C), lambda b, lo, hi: (0, 0)),
            pl.BlockSpec((1, C), lambda b, lo, hi: (0, 0)),
        ],
        out_specs=pl.BlockSpec((Q, C), lambda b, lo, hi: (b, 0)),
    )
    gat = pl.pallas_call(
        _gat_kernel,
        grid_spec=gat_spec,
        out_shape=jax.ShapeDtypeStruct((NP, C), jnp.float32),
    )

    for (Wl, bl, Wr, br, att, bias) in (
            (Wl0, bl0, Wr0, br0, att0, bias0),
            (Wl1, bl1, Wr1, br1, att1, bias1),
            (Wl2, bl2, Wr2, br2, att2, bias2)):
        x = gat(blo, bhi, src, x, x, Wl, bl[None, :], Wr, br[None, :],
                att.reshape(1, H * C), bias[None, :])

    gsum, cnt, res = pl.pallas_call(
        _pool_kernel,
        grid=(NB,),
        in_specs=[
            pl.BlockSpec((Q, C), lambda b: (b, 0)),
            pl.BlockSpec((1, 1, Q), lambda b: (b, 0, 0)),
            pl.BlockSpec((C, 1), lambda b: (0, 0)),
            pl.BlockSpec((1, 1), lambda b: (0, 0)),
        ],
        out_specs=[
            pl.BlockSpec((NG, C), lambda b: (0, 0)),
            pl.BlockSpec((NG, 1), lambda b: (0, 0)),
            pl.BlockSpec((NG, 1), lambda b: (0, 0)),
        ],
        out_shape=[
            jax.ShapeDtypeStruct((NG, C), jnp.float32),
            jax.ShapeDtypeStruct((NG, 1), jnp.float32),
            jax.ShapeDtypeStruct((NG, 1), jnp.float32),
        ],
    )(x, br3, fc_W, fc_b[None, :])

    return res[:, 0]


# speedup vs baseline: 1.0306x; 1.0306x over previous
import functools

import jax
import jax.numpy as jnp
from jax.experimental import pallas as pl
from jax.experimental.pallas import tpu as pltpu

N = 50000
NG = 512
H = 4
C = 64
K = 5
Q = 1024
NB = 49
NP = NB * Q
CC = 512
NCH = NP // CC

_BIG = 1e10
_INF = float("inf")
_IBIG = 2**30


def _knn_kernel(blo_ref, bhi_ref, posq_ref, bcol_ref, pch_ref, br_ref,
                src_ref):
    b = pl.program_id(0)
    qs = b * Q
    lo = blo_ref[b]
    hi = bhi_ref[b]
    c0 = lo // CC
    c1 = (hi - 1) // CC + 1

    qp = posq_ref[...]
    qx = qp[:, 0:1]
    qy = qp[:, 1:2]
    qz = qp[:, 2:3]
    qx2 = qx * qx + qy * qy + qz * qz
    qb = bcol_ref[...]
    rowg = jax.lax.broadcasted_iota(jnp.int32, (Q, CC), 0) + qs
    coll = jax.lax.broadcasted_iota(jnp.int32, (Q, CC), 1)
    pos10 = jax.lax.broadcasted_iota(jnp.int32, (Q, 2 * K), 1)

    def body(c, carry):
        best_d, best_i = carry
        base = c * CC
        cp = pch_ref[pl.ds(c, 1)].reshape(8, CC)
        cb = br_ref[pl.ds(c, 1), :]
        cx = cp[0:1, :]
        cy = cp[1:2, :]
        cz = cp[2:3, :]
        cx2 = cx * cx + cy * cy + cz * cz
        dot = jnp.dot(qp, cp, preferred_element_type=jnp.float32)
        d = cx2 - 2.0 * dot + qx2
        gidx = coll + base
        d = jnp.where((qb != cb) | (rowg == gidx), _BIG, d)
        cds, cis = [], []
        for _ in range(K):
            m = jnp.min(d, axis=1, keepdims=True)
            am = jnp.min(jnp.where(d == m, gidx, _IBIG), axis=1, keepdims=True)
            cds.append(m)
            cis.append(am)
            d = jnp.where(gidx == am, _INF, d)
        cat_d = jnp.concatenate([best_d] + cds, axis=1)
        cat_i = jnp.concatenate([best_i] + cis, axis=1)
        nds, nis = [], []
        for _ in range(K):
            m = jnp.min(cat_d, axis=1, keepdims=True)
            ap = jnp.min(jnp.where(cat_d == m, pos10, _IBIG), axis=1,
                         keepdims=True)
            sel = pos10 == ap
            nds.append(m)
            nis.append(jnp.sum(jnp.where(sel, cat_i, 0), axis=1,
                               keepdims=True))
            cat_d = jnp.where(sel, _INF, cat_d)
        return (jnp.concatenate(nds, axis=1), jnp.concatenate(nis, axis=1))

    init = (jnp.full((Q, K), _INF, jnp.float32),
            jnp.zeros((Q, K), jnp.int32))
    _, best_i = jax.lax.fori_loop(c0, c1, body, init)
    src_ref[...] = best_i


def _embed_kernel(z_ref, emb_ref, out_ref):
    zc = z_ref[...]
    cols = jax.lax.broadcasted_iota(jnp.int32, (Q, 128), 1)
    oh = (zc == cols).astype(jnp.float32)
    out_ref[...] = jnp.dot(oh, emb_ref[...],
                           preferred_element_type=jnp.float32)


def _gat_kernel(blo_ref, bhi_ref, src_ref, xq_ref, x_ref, Wl_ref, bl_ref,
                Wr_ref, br_ref, att_ref, bias_ref, out_ref):
    b = pl.program_id(0)
    lo = blo_ref[b]
    hi = bhi_ref[b]
    c0 = lo // CC
    c1 = (hi - 1) // CC + 1

    coll = jax.lax.broadcasted_iota(jnp.int32, (Q, CC), 1)

    def body(c, xjr):
        base = c * CC
        xc = x_ref[pl.ds(base, CC), :]
        new = []
        for k in range(K):
            idx = src_ref[:, k:k + 1] - base
            oh = (idx == coll).astype(jnp.float32)
            new.append(xjr[k] + jnp.dot(oh, xc,
                                        preferred_element_type=jnp.float32))
        return tuple(new)

    init = tuple(jnp.zeros((Q, C), jnp.float32) for _ in range(K))
    xjr = jax.lax.fori_loop(c0, c1, body, init)

    xi = jnp.dot(xq_ref[...], Wr_ref[...],
                 preferred_element_type=jnp.float32) + br_ref[...]
    att = att_ref[...]
    xjs, alphas = [], []
    for k in range(K):
        xjk = jnp.dot(xjr[k], Wl_ref[...],
                      preferred_element_type=jnp.float32) + bl_ref[...]
        e = xi + xjk
        e = jnp.where(e >= 0, e, 0.2 * e)
        p = e * att
        alphas.append([jnp.sum(p[:, h * C:(h + 1) * C], axis=1,
                               keepdims=True) for h in range(H)])
        xjs.append(xjk)

    acc = jnp.zeros((Q, C), jnp.float32)
    for h in range(H):
        ah = [alphas[k][h] for k in range(K)]
        amax = ah[0]
        for k in range(1, K):
            amax = jnp.maximum(amax, ah[k])
        ex = [jnp.exp(ah[k] - amax) for k in range(K)]
        den = ex[0]
        for k in range(1, K):
            den = den + ex[k]
        den = den + 1e-16
        for k in range(K):
            acc = acc + (ex[k] / den) * xjs[k][:, h * C:(h + 1) * C]

    xnew = acc * jnp.float32(1.0 / H) + bias_ref[...]
    out_ref[...] = jnp.maximum(xnew, 0.0)


def _pool_kernel(x_ref, br_ref, fcW_ref, fcb_ref, gsum_ref, cnt_ref,
                 res_ref):
    b = pl.program_id(0)

    @pl.when(b == 0)
    def _():
        gsum_ref[...] = jnp.zeros_like(gsum_ref)
        cnt_ref[...] = jnp.zeros_like(cnt_ref)

    brow = br_ref[0]
    gi = jax.lax.broadcasted_iota(jnp.int32, (NG, Q), 0)
    oh = (gi == brow).astype(jnp.float32)
    gsum_ref[...] += jnp.dot(oh, x_ref[...],
                             preferred_element_type=jnp.float32)
    cnt_ref[...] += jnp.sum(oh, axis=1, keepdims=True)

    @pl.when(b == NB - 1)
    def _():
        g = gsum_ref[...] / jnp.maximum(cnt_ref[...], 1.0)
        res_ref[...] = jnp.dot(g, fcW_ref[...],
                               preferred_element_type=jnp.float32) \
            + fcb_ref[...]


def kernel(z, pos, batch, emb, Wl0, bl0, Wr0, br0, att0, bias0,
           Wl1, bl1, Wr1, br1, att1, bias1,
           Wl2, bl2, Wr2, br2, att2, bias2, fc_W, fc_b):
    batch = batch.astype(jnp.int32)
    z = z.astype(jnp.int32)

    pad = NP - N
    posq = jnp.pad(pos, ((0, pad), (0, 5)))
    batp = jnp.pad(batch, (0, pad), constant_values=NG)
    bcol = batp[:, None]
    br3 = batp.reshape(NB, 1, Q)
    brch = batp.reshape(NCH, CC)
    pch = jnp.pad(pos, ((0, pad), (0, 5))).T.reshape(8, NCH, CC) \
        .transpose(1, 0, 2)
    zcol = jnp.pad(z, (0, pad))[:, None]

    ss = jnp.searchsorted(batch, jnp.arange(NG + 1)).astype(jnp.int32)
    firsts = jnp.minimum(jnp.arange(NB) * Q, N - 1)
    lasts = jnp.minimum(jnp.arange(NB) * Q + Q - 1, N - 1)
    blo = ss[batch[firsts]]
    bhi = ss[batch[lasts] + 1]

    knn_spec = pltpu.PrefetchScalarGridSpec(
        num_scalar_prefetch=2,
        grid=(NB,),
        in_specs=[
            pl.BlockSpec((Q, 8), lambda b, lo, hi: (b, 0)),
            pl.BlockSpec((Q, 1), lambda b, lo, hi: (b, 0)),
            pl.BlockSpec((NCH, 8, CC), lambda b, lo, hi: (0, 0, 0)),
            pl.BlockSpec((NCH, CC), lambda b, lo, hi: (0, 0)),
        ],
        out_specs=pl.BlockSpec((Q, K), lambda b, lo, hi: (b, 0)),
    )
    src = pl.pallas_call(
        _knn_kernel,
        grid_spec=knn_spec,
        out_shape=jax.ShapeDtypeStruct((NP, K), jnp.int32),
    )(blo, bhi, posq, bcol, pch, brch)

    embp = jnp.zeros((128, C), jnp.float32).at[:100].set(emb)
    x = pl.pallas_call(
        _embed_kernel,
        grid=(NB,),
        in_specs=[
            pl.BlockSpec((Q, 1), lambda b: (b, 0)),
            pl.BlockSpec((128, C), lambda b: (0, 0)),
        ],
        out_specs=pl.BlockSpec((Q, C), lambda b: (b, 0)),
        out_shape=jax.ShapeDtypeStruct((NP, C), jnp.float32),
    )(zcol, embp)

    gat_spec = pltpu.PrefetchScalarGridSpec(
        num_scalar_prefetch=2,
        grid=(NB,),
        in_specs=[
            pl.BlockSpec((Q, K), lambda b, lo, hi: (b, 0)),
            pl.BlockSpec((Q, C), lambda b, lo, hi: (b, 0)),
            pl.BlockSpec((NP, C), lambda b, lo, hi: (0, 0)),
            pl.BlockSpec((C, H * C), lambda b, lo, hi: (0, 0)),
            pl.BlockSpec((1, H * C), lambda b, lo, hi: (0, 0)),
            pl.BlockSpec((C, H * C), lambda b, lo, hi: (0, 0)),
            pl.BlockSpec((1, H * C), lambda b, lo, hi: (0, 0)),
            pl.BlockSpec((1, H * C), lambda b, lo, hi: (0, 0)),
            pl.BlockSpec((1, C), lambda b, lo, hi: (0, 0)),
        ],
        out_specs=pl.BlockSpec((Q, C), lambda b, lo, hi: (b, 0)),
    )
    gat = pl.pallas_call(
        _gat_kernel,
        grid_spec=gat_spec,
        out_shape=jax.ShapeDtypeStruct((NP, C), jnp.float32),
    )

    for (Wl, bl, Wr, br, att, bias) in (
            (Wl0, bl0, Wr0, br0, att0, bias0),
            (Wl1, bl1, Wr1, br1, att1, bias1),
            (Wl2, bl2, Wr2, br2, att2, bias2)):
        x = gat(blo, bhi, src, x, x, Wl, bl[None, :], Wr, br[None, :],
                att.reshape(1, H * C), bias[None, :])

    gsum, cnt, res = pl.pallas_call(
        _pool_kernel,
        grid=(NB,),
        in_specs=[
            pl.BlockSpec((Q, C), lambda b: (b, 0)),
            pl.BlockSpec((1, 1, Q), lambda b: (b, 0, 0)),
            pl.BlockSpec((C, 1), lambda b: (0, 0)),
            pl.BlockSpec((1, 1), lambda b: (0, 0)),
        ],
        out_specs=[
            pl.BlockSpec((NG, C), lambda b: (0, 0)),
            pl.BlockSpec((NG, 1), lambda b: (0, 0)),
            pl.BlockSpec((NG, 1), lambda b: (0, 0)),
        ],
        out_shape=[
            jax.ShapeDtypeStruct((NG, C), jnp.float32),
            jax.ShapeDtypeStruct((NG, 1), jnp.float32),
            jax.ShapeDtypeStruct((NG, 1), jnp.float32),
        ],
    )(x, br3, fc_W, fc_b[None, :])

    return res[:, 0]


# speedup vs baseline: 1.7730x; 1.7205x over previous
import functools

import jax
import jax.numpy as jnp
from jax.experimental import pallas as pl
from jax.experimental.pallas import tpu as pltpu
from jax.experimental.pallas import tpu_sc as plsc

N = 50000
NG = 512
H = 4
C = 64
K = 5
Q = 256
NB = 196
NP = NB * Q
CC = 512
NCH = NP // CC
W = 512
NBW = NP // W

_BIG = 1e10
_INF = float("inf")
_IBIG = 2**30


def _knn_kernel(blo_ref, bhi_ref, posq_ref, bcol_ref, pch_ref, br_ref,
                src_ref):
    b = pl.program_id(0)
    qs = b * Q
    lo = blo_ref[b]
    hi = bhi_ref[b]
    c0 = lo // CC
    c1 = (hi - 1) // CC + 1

    qp = posq_ref[...]
    qx = qp[:, 0:1]
    qy = qp[:, 1:2]
    qz = qp[:, 2:3]
    qx2 = qx * qx + qy * qy + qz * qz
    qb = bcol_ref[...]
    rowg = jax.lax.broadcasted_iota(jnp.int32, (Q, CC), 0) + qs
    coll = jax.lax.broadcasted_iota(jnp.int32, (Q, CC), 1)
    pos10 = jax.lax.broadcasted_iota(jnp.int32, (Q, 2 * K), 1)

    def body(c, carry):
        best_d, best_i = carry
        base = c * CC
        cp = pch_ref[pl.ds(c, 1)].reshape(8, CC)
        cb = br_ref[pl.ds(c, 1), :]
        cx = cp[0:1, :]
        cy = cp[1:2, :]
        cz = cp[2:3, :]
        cx2 = cx * cx + cy * cy + cz * cz
        dot = jnp.dot(qp, cp, preferred_element_type=jnp.float32)
        d = cx2 - 2.0 * dot + qx2
        gidx = coll + base
        d = jnp.where((qb != cb) | (rowg == gidx), _BIG, d)
        cds, cis = [], []
        for _ in range(K):
            m = jnp.min(d, axis=1, keepdims=True)
            am = jnp.min(jnp.where(d == m, gidx, _IBIG), axis=1, keepdims=True)
            cds.append(m)
            cis.append(am)
            d = jnp.where(gidx == am, _INF, d)
        cat_d = jnp.concatenate([best_d] + cds, axis=1)
        cat_i = jnp.concatenate([best_i] + cis, axis=1)
        nds, nis = [], []
        for _ in range(K):
            m = jnp.min(cat_d, axis=1, keepdims=True)
            ap = jnp.min(jnp.where(cat_d == m, pos10, _IBIG), axis=1,
                         keepdims=True)
            sel = pos10 == ap
            nds.append(m)
            nis.append(jnp.sum(jnp.where(sel, cat_i, 0), axis=1,
                               keepdims=True))
            cat_d = jnp.where(sel, _INF, cat_d)
        return (jnp.concatenate(nds, axis=1), jnp.concatenate(nis, axis=1))

    init = (jnp.full((Q, K), _INF, jnp.float32),
            jnp.zeros((Q, K), jnp.int32))
    _, best_i = jax.lax.fori_loop(c0, c1, body, init)
    src_ref[...] = best_i


def _embed_kernel(z_ref, emb_ref, out_ref, out_nm_ref):
    zr = z_ref[0]
    rows = jax.lax.broadcasted_iota(jnp.int32, (128, W), 0)
    oh = (zr == rows).astype(jnp.float32)
    xw = jnp.dot(emb_ref[...], oh,
                 preferred_element_type=jnp.float32)
    out_ref[0] = xw
    out_nm_ref[:, :C] = xw.T


EP = K * NP
SC_NC = 2
SC_NS = 16
SC_NW = SC_NC * SC_NS
BPW = EP // SC_NW
GCH = 280
GNC = BPW // GCH


def _sc_gather(x_nm, srcf):
    mesh = plsc.VectorSubcoreMesh(core_axis_name="c", subcore_axis_name="s")

    @functools.partial(
        pl.kernel, mesh=mesh,
        out_type=jax.ShapeDtypeStruct((EP, 128), jnp.float32),
        scratch_types=[
            pltpu.VMEM((GCH,), jnp.int32),
            pltpu.VMEM((GCH, 128), jnp.float32),
            pltpu.SemaphoreType.DMA,
        ],
    )
    def k(x_hbm, idx_hbm, out_hbm, idx_v, rows_v, sem):
        wid = jax.lax.axis_index("s") * SC_NC + jax.lax.axis_index("c")
        base0 = wid * BPW

        def body(j, carry):
            base = base0 + j * GCH
            pltpu.sync_copy(idx_hbm.at[pl.ds(base, GCH)], idx_v)
            pltpu.async_copy(x_hbm.at[idx_v], rows_v, sem).wait()
            pltpu.sync_copy(rows_v, out_hbm.at[pl.ds(base, GCH)])
            return carry

        jax.lax.fori_loop(0, GNC, body, 0)

    return k(x_nm, srcf)


def _gat_sc_kernel(xq_ref, xg_ref, WlT_ref, blT_ref, WrT_ref, brT_ref,
                   attB_ref, biasT_ref, out_ref, out_nm_ref):
    xq = xq_ref[0]
    xi = jnp.dot(WrT_ref[...], xq,
                 preferred_element_type=jnp.float32) + brT_ref[...]
    attB = attB_ref[...]
    xjs, alphas = [], []
    for k in range(K):
        xj_wc = xg_ref[k, 0][:, :C]
        xjk = jax.lax.dot_general(
            WlT_ref[...], xj_wc, (((1,), (1,)), ((), ())),
            preferred_element_type=jnp.float32) + blT_ref[...]
        e = xi + xjk
        e = jnp.where(e >= 0, e, 0.2 * e)
        alphas.append(jnp.dot(attB, e,
                              preferred_element_type=jnp.float32))
        xjs.append(xjk)

    acc = jnp.zeros((C, W), jnp.float32)
    for h in range(H):
        ah = [alphas[k][h:h + 1, :] for k in range(K)]
        amax = ah[0]
        for k in range(1, K):
            amax = jnp.maximum(amax, ah[k])
        ex = [jnp.exp(ah[k] - amax) for k in range(K)]
        den = ex[0]
        for k in range(1, K):
            den = den + ex[k]
        den = den + 1e-16
        for k in range(K):
            acc = acc + (ex[k] / den) * xjs[k][h * C:(h + 1) * C, :]

    xnew = acc * jnp.float32(1.0 / H) + biasT_ref[...]
    xnew = jnp.maximum(xnew, 0.0)
    out_ref[0] = xnew
    out_nm_ref[:, :C] = xnew.T


def _pool_kernel(x_ref, bq_ref, fcWT_ref, fcb_ref, gsum_ref, cnt_ref,
                 res_ref):
    b = pl.program_id(0)

    @pl.when(b == 0)
    def _():
        gsum_ref[...] = jnp.zeros_like(gsum_ref)
        cnt_ref[...] = jnp.zeros_like(cnt_ref)

    bq = bq_ref[...]
    gl = jax.lax.broadcasted_iota(jnp.int32, (W, NG), 1)
    oh = (bq == gl).astype(jnp.float32)
    gsum_ref[...] += jnp.dot(x_ref[0], oh,
                             preferred_element_type=jnp.float32)
    cnt_ref[...] += jnp.sum(oh, axis=0, keepdims=True)

    @pl.when(b == NBW - 1)
    def _():
        g = gsum_ref[...] / jnp.maximum(cnt_ref[...], 1.0)
        r = jnp.dot(fcWT_ref[...], g,
                    preferred_element_type=jnp.float32)
        res_ref[...] = r[0:1, :] + fcb_ref[...]


def kernel(z, pos, batch, emb, Wl0, bl0, Wr0, br0, att0, bias0,
           Wl1, bl1, Wr1, br1, att1, bias1,
           Wl2, bl2, Wr2, br2, att2, bias2, fc_W, fc_b):
    batch = batch.astype(jnp.int32)
    z = z.astype(jnp.int32)

    pad = NP - N
    posq = jnp.pad(pos, ((0, pad), (0, 5)))
    batp = jnp.pad(batch, (0, pad), constant_values=NG)
    bcol = batp[:, None]
    br3 = batp.reshape(NB, 1, Q)
    brch = batp.reshape(NCH, CC)
    pch = jnp.pad(pos, ((0, pad), (0, 5))).T.reshape(8, NCH, CC) \
        .transpose(1, 0, 2)
    zcol = jnp.pad(z, (0, pad))[:, None]

    ss = jnp.searchsorted(batch, jnp.arange(NG + 1)).astype(jnp.int32)
    firsts = jnp.minimum(jnp.arange(NB) * Q, N - 1)
    lasts = jnp.minimum(jnp.arange(NB) * Q + Q - 1, N - 1)
    blo = ss[batch[firsts]]
    bhi = ss[batch[lasts] + 1]

    knn_spec = pltpu.PrefetchScalarGridSpec(
        num_scalar_prefetch=2,
        grid=(NB,),
        in_specs=[
            pl.BlockSpec((Q, 8), lambda b, lo, hi: (b, 0)),
            pl.BlockSpec((Q, 1), lambda b, lo, hi: (b, 0)),
            pl.BlockSpec((NCH, 8, CC), lambda b, lo, hi: (0, 0, 0)),
            pl.BlockSpec((NCH, CC), lambda b, lo, hi: (0, 0)),
        ],
        out_specs=pl.BlockSpec((Q, K), lambda b, lo, hi: (b, 0)),
    )
    src = pl.pallas_call(
        _knn_kernel,
        grid_spec=knn_spec,
        out_shape=jax.ShapeDtypeStruct((NP, K), jnp.int32),
    )(blo, bhi, posq, bcol, pch, brch)

    srcT = src.T
    zrow = jnp.pad(z, (0, pad)).reshape(NBW, 1, W)

    embT = jnp.zeros((C, 128), jnp.float32).at[:, :100].set(emb.T)
    x, x_nm = pl.pallas_call(
        _embed_kernel,
        grid=(NBW,),
        in_specs=[
            pl.BlockSpec((1, 1, W), lambda b: (b, 0, 0)),
            pl.BlockSpec((C, 128), lambda b: (0, 0)),
        ],
        out_specs=[
            pl.BlockSpec((1, C, W), lambda b: (b, 0, 0)),
            pl.BlockSpec((W, 128), lambda b: (b, 0)),
        ],
        out_shape=[
            jax.ShapeDtypeStruct((NBW, C, W), jnp.float32),
            jax.ShapeDtypeStruct((NP, 128), jnp.float32),
        ],
    )(zrow, embT)

    gat_sc = pl.pallas_call(
        _gat_sc_kernel,
        grid=(NBW,),
        in_specs=[
            pl.BlockSpec((1, C, W), lambda b: (b, 0, 0)),
            pl.BlockSpec((K, 1, W, 128), lambda b: (0, b, 0, 0)),
            pl.BlockSpec((H * C, C), lambda b: (0, 0)),
            pl.BlockSpec((H * C, 1), lambda b: (0, 0)),
            pl.BlockSpec((H * C, C), lambda b: (0, 0)),
            pl.BlockSpec((H * C, 1), lambda b: (0, 0)),
            pl.BlockSpec((8, H * C), lambda b: (0, 0)),
            pl.BlockSpec((C, 1), lambda b: (0, 0)),
        ],
        out_specs=[
            pl.BlockSpec((1, C, W), lambda b: (b, 0, 0)),
            pl.BlockSpec((W, 128), lambda b: (b, 0)),
        ],
        out_shape=[
            jax.ShapeDtypeStruct((NBW, C, W), jnp.float32),
            jax.ShapeDtypeStruct((NP, 128), jnp.float32),
        ],
    )

    srcf = srcT.reshape(EP)
    eyeH = jnp.eye(H, dtype=jnp.float32)
    for (Wl, bl, Wr, br, att, bias) in (
            (Wl0, bl0, Wr0, br0, att0, bias0),
            (Wl1, bl1, Wr1, br1, att1, bias1),
            (Wl2, bl2, Wr2, br2, att2, bias2)):
        attB = (eyeH[:, :, None] * att[None, :, :]).reshape(H, H * C)
        attB = jnp.pad(attB, ((0, 4), (0, 0)))
        xg = _sc_gather(x_nm, srcf).reshape(K, NBW, W, 128)
        x, x_nm = gat_sc(x, xg, Wl.T, bl[:, None], Wr.T, br[:, None],
                         attB, bias[:, None])

    gsum, cnt, res = pl.pallas_call(
        _pool_kernel,
        grid=(NBW,),
        in_specs=[
            pl.BlockSpec((1, C, W), lambda b: (b, 0, 0)),
            pl.BlockSpec((W, 1), lambda b: (b, 0)),
            pl.BlockSpec((8, C), lambda b: (0, 0)),
            pl.BlockSpec((1, 1), lambda b: (0, 0)),
        ],
        out_specs=[
            pl.BlockSpec((C, NG), lambda b: (0, 0)),
            pl.BlockSpec((1, NG), lambda b: (0, 0)),
            pl.BlockSpec((1, NG), lambda b: (0, 0)),
        ],
        out_shape=[
            jax.ShapeDtypeStruct((C, NG), jnp.float32),
            jax.ShapeDtypeStruct((1, NG), jnp.float32),
            jax.ShapeDtypeStruct((1, NG), jnp.float32),
        ],
    )(x, bcol, jnp.pad(fc_W.T, ((0, 7), (0, 0))), fc_b[None, :])

    return res[0]
